# baseline (speedup 1.0000x reference)
import functools

import jax
import jax.numpy as jnp
from jax import lax
from jax.experimental import pallas as pl
from jax.experimental.pallas import tpu as pltpu
from jax.experimental.pallas import tpu_sc as plsc

_RATIO = 0.25
_R = 0.2
_NS = 32
_EPS = 1e-5

_LANES = 1280


def _fps_pallas(px, py, pz, n, m, rows):
    def fps_kernel(px_ref, py_ref, pz_ref, out_ref):
        pxv = px_ref[...]
        pyv = py_ref[...]
        pzv = pz_ref[...]
        r = jax.lax.broadcasted_iota(jnp.int32, (rows, _LANES), 0)
        c = jax.lax.broadcasted_iota(jnp.int32, (rows, _LANES), 1)
        flat = r * _LANES + c
        in_range = flat < n
        qx = pxv[0, 0]
        qy = pyv[0, 0]
        qz = pzv[0, 0]
        dx = pxv - qx
        dy = pyv - qy
        dz = pzv - qz
        d0 = (dx * dx + dy * dy) + dz * dz
        dist = jnp.where(in_range, d0, -jnp.inf)
        out_ref[0, 0] = 0

        big = jnp.int32(rows * _LANES)

        def body(i, dist):
            mv = jnp.max(dist)
            sel = dist == mv
            nxt = jnp.min(jnp.where(sel, flat, big)).astype(jnp.int32)
            hi = jnp.max(jnp.where(sel, flat, -1)).astype(jnp.int32)
            qx = jnp.max(jnp.where(sel, pxv, -jnp.inf))
            qy = jnp.max(jnp.where(sel, pyv, -jnp.inf))
            qz = jnp.max(jnp.where(sel, pzv, -jnp.inf))

            def tie_exact():
                s2 = flat == nxt
                return (jnp.max(jnp.where(s2, pxv, -jnp.inf)),
                        jnp.max(jnp.where(s2, pyv, -jnp.inf)),
                        jnp.max(jnp.where(s2, pzv, -jnp.inf)))

            qx, qy, qz = jax.lax.cond(
                hi == nxt, lambda: (qx, qy, qz), tie_exact)
            dx = pxv - qx
            dy = pyv - qy
            dz = pzv - qz
            d = (dx * dx + dy * dy) + dz * dz
            out_ref[0, i] = nxt
            return jnp.where(in_range, jnp.minimum(dist, d), -jnp.inf)

        jax.lax.fori_loop(1, m, body, dist)

    return pl.pallas_call(
        fps_kernel,
        out_shape=jax.ShapeDtypeStruct((1, m), jnp.int32),
        in_specs=[pl.BlockSpec(memory_space=pltpu.VMEM)] * 3,
        out_specs=pl.BlockSpec(memory_space=pltpu.SMEM),
    )(px, py, pz)


def _ball_query_pallas(cp8, p8t, n, npad, nblocks, bc):
    r2 = _R * _R

    def bq_kernel(cp_ref, pt_ref, nbr_ref, d_ref):
        cp = cp_ref[0]
        pt = pt_ref[...]
        cn = jnp.sum(cp * cp, axis=1)[:, None]
        pn = jnp.sum(pt * pt, axis=0)[None, :]
        g = jnp.dot(cp, pt, preferred_element_type=jnp.float32)
        d2 = cn + pn - 2.0 * g
        d2 = jnp.maximum(d2, 0.0)
        cols = jax.lax.broadcasted_iota(jnp.int32, (bc, npad), 1)
        d2 = jnp.where((d2 <= r2) & (cols < n), d2, jnp.inf)
        big = jnp.int32(npad)
        mprev = jnp.full((bc,), -1.0, jnp.float32)
        cprev = jnp.full((bc,), -1, jnp.int32)
        for k in range(_NS):
            succ = (d2 > mprev[:, None]) | (
                (d2 == mprev[:, None]) & (cols > cprev[:, None]))
            rowmin = jnp.min(jnp.where(succ, d2, jnp.inf), axis=1)
            okrow = rowmin > mprev
            amin = jnp.min(
                jnp.where((d2 == rowmin[:, None])
                          & (okrow[:, None] | (cols > cprev[:, None])),
                          cols, big), axis=1).astype(jnp.int32)
            nbr_ref[0, :, k] = amin
            d_ref[0, :, k] = rowmin
            mprev = rowmin
            cprev = amin

    return pl.pallas_call(
        bq_kernel,
        grid=(nblocks,),
        compiler_params=pltpu.CompilerParams(
            dimension_semantics=("parallel",)),
        in_specs=[
            pl.BlockSpec((1, bc, 8), lambda i: (i, 0, 0)),
            pl.BlockSpec((8, npad), lambda i: (0, 0)),
        ],
        out_specs=[
            pl.BlockSpec((1, bc, _NS), lambda i: (i, 0, 0)),
            pl.BlockSpec((1, bc, _NS), lambda i: (i, 0, 0)),
        ],
        out_shape=[
            jax.ShapeDtypeStruct((nblocks, bc, _NS), jnp.int32),
            jax.ShapeDtypeStruct((nblocks, bc, _NS), jnp.float32),
        ],
    )(cp8, p8t)


def _sc_gather(tab, idxf, B, D):
    NW = 32
    bpw = B // NW
    CH = 256
    nch = bpw // CH
    mesh = plsc.VectorSubcoreMesh(core_axis_name="c", subcore_axis_name="s")

    @functools.partial(
        pl.kernel,
        mesh=mesh,
        out_type=jax.ShapeDtypeStruct((B, D), jnp.float32),
        scratch_types=[
            pltpu.VMEM((bpw,), jnp.int32),
            pltpu.VMEM((CH, D), jnp.float32),
            pltpu.VMEM((CH, D), jnp.float32),
            pltpu.SemaphoreType.DMA,
            pltpu.SemaphoreType.DMA,
        ],
    )
    def gk(tab_hbm, idx_hbm, out_hbm, idx_v, buf0, buf1, s0, s1):
        wid = lax.axis_index("s") * 2 + lax.axis_index("c")
        base = wid * bpw
        pltpu.sync_copy(idx_hbm.at[pl.ds(base, bpw)], idx_v)
        bufs = (buf0, buf1)
        sems = (s0, s1)
        cp = pltpu.async_copy(tab_hbm.at[idx_v.at[pl.ds(0, CH)]], buf0, s0)
        for ch in range(nch):
            nxt = None
            if ch + 1 < nch:
                nxt = pltpu.async_copy(
                    tab_hbm.at[idx_v.at[pl.ds((ch + 1) * CH, CH)]],
                    bufs[(ch + 1) % 2], sems[(ch + 1) % 2])
            cp.wait()
            pltpu.sync_copy(bufs[ch % 2],
                            out_hbm.at[pl.ds(base + ch * CH, CH)])
            cp = nxt

    return gk(tab, idxf)


def _mlp_pool_pallas(xj, relx, rely, relz, vmask, A1, p1x, p1y, p1z, c1,
                     A2, c2, nrows, bl):
    nb = nrows // bl
    gc = bl // _NS

    def mlp_kernel(xj_ref, rx_ref, ry_ref, rz_ref, vm_ref, a1_ref,
                   p1x_ref, p1y_ref, p1z_ref, c1_ref, a2_ref, c2_ref,
                   out_ref):
        xjv = xj_ref[...]
        z1 = jnp.dot(xjv, a1_ref[...], preferred_element_type=jnp.float32)
        z1 = z1 + rx_ref[...] * p1x_ref[...]
        z1 = z1 + ry_ref[...] * p1y_ref[...]
        z1 = z1 + rz_ref[...] * p1z_ref[...]
        h1 = jnp.maximum(z1 + c1_ref[...], 0.0)
        z2 = jnp.dot(h1, a2_ref[...], preferred_element_type=jnp.float32)
        h2 = jnp.maximum(z2 + c2_ref[...], 0.0)
        vm = vm_ref[...] > 0.0
        h2 = jnp.where(vm, h2, -jnp.inf)
        h2 = h2.reshape(gc, _NS, 128)
        mx = jnp.max(h2, axis=1)
        anyv = jnp.max(jnp.where(vm, 1.0, 0.0).reshape(gc, _NS, 1), axis=1)
        out_ref[...] = jnp.where(anyv > 0.0, mx, 0.0)

    return pl.pallas_call(
        mlp_kernel,
        grid=(nb,),
        compiler_params=pltpu.CompilerParams(
            dimension_semantics=("parallel",)),
        in_specs=[
            pl.BlockSpec((bl, 128), lambda i: (i, 0)),
            pl.BlockSpec((bl, 1), lambda i: (i, 0)),
            pl.BlockSpec((bl, 1), lambda i: (i, 0)),
            pl.BlockSpec((bl, 1), lambda i: (i, 0)),
            pl.BlockSpec((bl, 1), lambda i: (i, 0)),
            pl.BlockSpec((128, 128), lambda i: (0, 0)),
            pl.BlockSpec((1, 128), lambda i: (0, 0)),
            pl.BlockSpec((1, 128), lambda i: (0, 0)),
            pl.BlockSpec((1, 128), lambda i: (0, 0)),
            pl.BlockSpec((1, 128), lambda i: (0, 0)),
            pl.BlockSpec((128, 128), lambda i: (0, 0)),
            pl.BlockSpec((1, 128), lambda i: (0, 0)),
        ],
        out_specs=pl.BlockSpec((gc, 128), lambda i: (i, 0)),
        out_shape=jax.ShapeDtypeStruct((nrows // _NS, 128), jnp.float32),
    )(xj, relx, rely, relz, vmask, A1, p1x, p1y, p1z, c1, A2, c2)


def kernel(x, pos, batch, W1, g1, b1, rm1, rv1, W2, g2, b2, rm2, rv2):
    N, C = x.shape
    m = int(N * _RATIO)

    npad = ((N + _LANES - 1) // _LANES) * _LANES
    rows = npad // _LANES
    p = jnp.pad(pos, ((0, npad - N), (0, 0)))
    px = p[:, 0].reshape(rows, _LANES)
    py = p[:, 1].reshape(rows, _LANES)
    pz = p[:, 2].reshape(rows, _LANES)
    idx = _fps_pallas(px, py, pz, N, m, rows)[0]

    cent = pos[idx]

    bc = 512
    mpad = ((m + bc - 1) // bc) * bc
    nblocks = mpad // bc
    cp8 = jnp.zeros((mpad, 8), jnp.float32).at[:m, :3].set(cent)
    cp8 = cp8.reshape(nblocks, bc, 8)
    p8t = jnp.zeros((8, npad), jnp.float32).at[:3, :N].set(pos.T)
    nbr_b, d_b = _ball_query_pallas(cp8, p8t, N, npad, nblocks, bc)
    nbr = nbr_b.reshape(mpad, _NS)[:m]
    dsel = d_b.reshape(mpad, _NS)[:m]
    nbr = jnp.minimum(nbr, N - 1)
    valid = jnp.isfinite(dsel)

    s1 = g1 / jnp.sqrt(rv1 + _EPS)
    c1 = (b1 - rm1 * s1)[None, :]
    W1s = W1 * s1[:, None]
    A1 = W1s[:, :C].T
    p1 = W1s[:, C:].T
    s2 = g2 / jnp.sqrt(rv2 + _EPS)
    c2 = (b2 - rm2 * s2)[None, :]
    A2 = (W2 * s2[:, None]).T

    bl = 2048
    nrows = mpad * _NS
    nbr_flat = jnp.zeros((nrows,), jnp.int32).at[: m * _NS].set(
        nbr.reshape(-1))
    vm_flat = jnp.zeros((nrows,), jnp.float32).at[: m * _NS].set(
        valid.reshape(-1).astype(jnp.float32))
    g = _sc_gather(x, nbr_flat, nrows, 128)
    pj = pos[nbr_flat]
    centr = jnp.zeros((mpad, 3), jnp.float32).at[:m].set(cent)
    rel = pj - jnp.repeat(centr, _NS, axis=0)
    out = _mlp_pool_pallas(
        g, rel[:, 0:1], rel[:, 1:2], rel[:, 2:3], vm_flat[:, None],
        A1, p1[0:1], p1[1:2], p1[2:3], c1, A2, c2, nrows, bl)
    x_out = out[:m]
    return x_out, pos[idx], batch[idx]


# speedup vs baseline: 1.4201x; 1.4201x over previous
import functools

import jax
import jax.numpy as jnp
from jax import lax
from jax.experimental import pallas as pl
from jax.experimental.pallas import tpu as pltpu
from jax.experimental.pallas import tpu_sc as plsc

_RATIO = 0.25
_R = 0.2
_NS = 32
_EPS = 1e-5

_LANES = 1280


def _fps_pallas(px, py, pz, n, m, rows):
    def fps_kernel(px_ref, py_ref, pz_ref, out_ref):
        pxv = px_ref[...]
        pyv = py_ref[...]
        pzv = pz_ref[...]
        r = jax.lax.broadcasted_iota(jnp.int32, (rows, _LANES), 0)
        c = jax.lax.broadcasted_iota(jnp.int32, (rows, _LANES), 1)
        flat = r * _LANES + c
        in_range = flat < n
        qx = pxv[0, 0]
        qy = pyv[0, 0]
        qz = pzv[0, 0]
        dx = pxv - qx
        dy = pyv - qy
        dz = pzv - qz
        d0 = (dx * dx + dy * dy) + dz * dz
        dist = jnp.where(in_range, d0, -jnp.inf)
        out_ref[0, 0] = 0

        big = jnp.int32(rows * _LANES)

        def body(i, dist):
            mv = jnp.max(dist)
            sel = dist == mv
            nxt = jnp.min(jnp.where(sel, flat, big)).astype(jnp.int32)
            hi = jnp.max(jnp.where(sel, flat, -1)).astype(jnp.int32)
            qx = jnp.max(jnp.where(sel, pxv, -jnp.inf))
            qy = jnp.max(jnp.where(sel, pyv, -jnp.inf))
            qz = jnp.max(jnp.where(sel, pzv, -jnp.inf))

            def tie_exact():
                s2 = flat == nxt
                return (jnp.max(jnp.where(s2, pxv, -jnp.inf)),
                        jnp.max(jnp.where(s2, pyv, -jnp.inf)),
                        jnp.max(jnp.where(s2, pzv, -jnp.inf)))

            qx, qy, qz = jax.lax.cond(
                hi == nxt, lambda: (qx, qy, qz), tie_exact)
            dx = pxv - qx
            dy = pyv - qy
            dz = pzv - qz
            d = (dx * dx + dy * dy) + dz * dz
            out_ref[0, i] = nxt
            return jnp.where(in_range, jnp.minimum(dist, d), -jnp.inf)

        jax.lax.fori_loop(1, m, body, dist)

    return pl.pallas_call(
        fps_kernel,
        out_shape=jax.ShapeDtypeStruct((1, m), jnp.int32),
        in_specs=[pl.BlockSpec(memory_space=pltpu.VMEM)] * 3,
        out_specs=pl.BlockSpec(memory_space=pltpu.SMEM),
    )(px, py, pz)


def _ball_query_pallas(cp8, p8t, n, npad, nblocks, bc):
    r2 = _R * _R

    def bq_kernel(cp_ref, pt_ref, nbr_ref, d_ref):
        cp = cp_ref[0]
        pt = pt_ref[...]
        cn = jnp.sum(cp * cp, axis=1)[:, None]
        pn = jnp.sum(pt * pt, axis=0)[None, :]
        g = jnp.dot(cp, pt, preferred_element_type=jnp.float32)
        d2 = cn + pn - 2.0 * g
        d2 = jnp.maximum(d2, 0.0)
        cols = jax.lax.broadcasted_iota(jnp.int32, (bc, npad), 1)
        d2 = jnp.where((d2 <= r2) & (cols < n), d2, jnp.inf)
        big = jnp.int32(npad)
        for k in range(_NS):
            rowmin = jnp.min(d2, axis=1)
            eq = d2 == rowmin[:, None]
            amin = jnp.min(jnp.where(eq, cols, big), axis=1).astype(jnp.int32)
            nbr_ref[0, :, k] = amin
            d_ref[0, :, k] = rowmin
            d2 = jnp.where(eq, jnp.inf, d2)

    return pl.pallas_call(
        bq_kernel,
        grid=(nblocks,),
        compiler_params=pltpu.CompilerParams(
            dimension_semantics=("parallel",)),
        in_specs=[
            pl.BlockSpec((1, bc, 8), lambda i: (i, 0, 0)),
            pl.BlockSpec((8, npad), lambda i: (0, 0)),
        ],
        out_specs=[
            pl.BlockSpec((1, bc, _NS), lambda i: (i, 0, 0)),
            pl.BlockSpec((1, bc, _NS), lambda i: (i, 0, 0)),
        ],
        out_shape=[
            jax.ShapeDtypeStruct((nblocks, bc, _NS), jnp.int32),
            jax.ShapeDtypeStruct((nblocks, bc, _NS), jnp.float32),
        ],
    )(cp8, p8t)


def _sc_gather(tab, idxf, B, D):
    NW = 32
    bpw = B // NW
    CH = 256
    nch = bpw // CH
    mesh = plsc.VectorSubcoreMesh(core_axis_name="c", subcore_axis_name="s")

    @functools.partial(
        pl.kernel,
        mesh=mesh,
        out_type=jax.ShapeDtypeStruct((B, D), jnp.float32),
        scratch_types=[
            pltpu.VMEM((bpw,), jnp.int32),
            pltpu.VMEM((CH, D), jnp.float32),
            pltpu.VMEM((CH, D), jnp.float32),
            pltpu.SemaphoreType.DMA,
            pltpu.SemaphoreType.DMA,
        ],
    )
    def gk(tab_hbm, idx_hbm, out_hbm, idx_v, buf0, buf1, s0, s1):
        wid = lax.axis_index("s") * 2 + lax.axis_index("c")
        base = wid * bpw
        pltpu.sync_copy(idx_hbm.at[pl.ds(base, bpw)], idx_v)
        bufs = (buf0, buf1)
        sems = (s0, s1)
        cp = pltpu.async_copy(tab_hbm.at[idx_v.at[pl.ds(0, CH)]], buf0, s0)
        for ch in range(nch):
            nxt = None
            if ch + 1 < nch:
                nxt = pltpu.async_copy(
                    tab_hbm.at[idx_v.at[pl.ds((ch + 1) * CH, CH)]],
                    bufs[(ch + 1) % 2], sems[(ch + 1) % 2])
            cp.wait()
            pltpu.sync_copy(bufs[ch % 2],
                            out_hbm.at[pl.ds(base + ch * CH, CH)])
            cp = nxt

    return gk(tab, idxf)


def _mlp_pool_pallas(xj, relx, rely, relz, vmask, A1, p1x, p1y, p1z, c1,
                     A2, c2, nrows, bl):
    nb = nrows // bl
    gc = bl // _NS

    def mlp_kernel(xj_ref, rx_ref, ry_ref, rz_ref, vm_ref, a1_ref,
                   p1x_ref, p1y_ref, p1z_ref, c1_ref, a2_ref, c2_ref,
                   out_ref):
        xjv = xj_ref[...]
        z1 = jnp.dot(xjv, a1_ref[...], preferred_element_type=jnp.float32)
        z1 = z1 + rx_ref[...] * p1x_ref[...]
        z1 = z1 + ry_ref[...] * p1y_ref[...]
        z1 = z1 + rz_ref[...] * p1z_ref[...]
        h1 = jnp.maximum(z1 + c1_ref[...], 0.0)
        z2 = jnp.dot(h1, a2_ref[...], preferred_element_type=jnp.float32)
        h2 = jnp.maximum(z2 + c2_ref[...], 0.0)
        vm = vm_ref[...] > 0.0
        h2 = jnp.where(vm, h2, -jnp.inf)
        h2 = h2.reshape(gc, _NS, 128)
        mx = jnp.max(h2, axis=1)
        anyv = jnp.max(jnp.where(vm, 1.0, 0.0).reshape(gc, _NS, 1), axis=1)
        out_ref[...] = jnp.where(anyv > 0.0, mx, 0.0)

    return pl.pallas_call(
        mlp_kernel,
        grid=(nb,),
        compiler_params=pltpu.CompilerParams(
            dimension_semantics=("parallel",)),
        in_specs=[
            pl.BlockSpec((bl, 128), lambda i: (i, 0)),
            pl.BlockSpec((bl, 1), lambda i: (i, 0)),
            pl.BlockSpec((bl, 1), lambda i: (i, 0)),
            pl.BlockSpec((bl, 1), lambda i: (i, 0)),
            pl.BlockSpec((bl, 1), lambda i: (i, 0)),
            pl.BlockSpec((128, 128), lambda i: (0, 0)),
            pl.BlockSpec((1, 128), lambda i: (0, 0)),
            pl.BlockSpec((1, 128), lambda i: (0, 0)),
            pl.BlockSpec((1, 128), lambda i: (0, 0)),
            pl.BlockSpec((1, 128), lambda i: (0, 0)),
            pl.BlockSpec((128, 128), lambda i: (0, 0)),
            pl.BlockSpec((1, 128), lambda i: (0, 0)),
        ],
        out_specs=pl.BlockSpec((gc, 128), lambda i: (i, 0)),
        out_shape=jax.ShapeDtypeStruct((nrows // _NS, 128), jnp.float32),
    )(xj, relx, rely, relz, vmask, A1, p1x, p1y, p1z, c1, A2, c2)


def kernel(x, pos, batch, W1, g1, b1, rm1, rv1, W2, g2, b2, rm2, rv2):
    N, C = x.shape
    m = int(N * _RATIO)

    npad = ((N + _LANES - 1) // _LANES) * _LANES
    rows = npad // _LANES
    p = jnp.pad(pos, ((0, npad - N), (0, 0)))
    px = p[:, 0].reshape(rows, _LANES)
    py = p[:, 1].reshape(rows, _LANES)
    pz = p[:, 2].reshape(rows, _LANES)
    idx = _fps_pallas(px, py, pz, N, m, rows)[0]

    cent = pos[idx]

    bc = 512
    mpad = ((m + bc - 1) // bc) * bc
    nblocks = mpad // bc
    cp8 = jnp.zeros((mpad, 8), jnp.float32).at[:m, :3].set(cent)
    cp8 = cp8.reshape(nblocks, bc, 8)
    p8t = jnp.zeros((8, npad), jnp.float32).at[:3, :N].set(pos.T)
    nbr_b, d_b = _ball_query_pallas(cp8, p8t, N, npad, nblocks, bc)
    nbr = nbr_b.reshape(mpad, _NS)[:m]
    dsel = d_b.reshape(mpad, _NS)[:m]
    nbr = jnp.minimum(nbr, N - 1)
    valid = jnp.isfinite(dsel)

    s1 = g1 / jnp.sqrt(rv1 + _EPS)
    c1 = (b1 - rm1 * s1)[None, :]
    W1s = W1 * s1[:, None]
    A1 = W1s[:, :C].T
    p1 = W1s[:, C:].T
    s2 = g2 / jnp.sqrt(rv2 + _EPS)
    c2 = (b2 - rm2 * s2)[None, :]
    A2 = (W2 * s2[:, None]).T

    bl = 2048
    nrows = mpad * _NS
    nbr_flat = jnp.zeros((nrows,), jnp.int32).at[: m * _NS].set(
        nbr.reshape(-1))
    vm_flat = jnp.zeros((nrows,), jnp.float32).at[: m * _NS].set(
        valid.reshape(-1).astype(jnp.float32))
    g = _sc_gather(x, nbr_flat, nrows, 128)
    pj = pos[nbr_flat]
    centr = jnp.zeros((mpad, 3), jnp.float32).at[:m].set(cent)
    rel = pj - jnp.repeat(centr, _NS, axis=0)
    out = _mlp_pool_pallas(
        g, rel[:, 0:1], rel[:, 1:2], rel[:, 2:3], vm_flat[:, None],
        A1, p1[0:1], p1[1:2], p1[2:3], c1, A2, c2, nrows, bl)
    x_out = out[:m]
    return x_out, pos[idx], batch[idx]
